# Initial kernel scaffold; baseline (speedup 1.0000x reference)
import jax
import jax.numpy as jnp
from jax.experimental import pallas as pl


def kernel(x, W, b):
    raise NotImplementedError("write your pallas kernel here")


# speedup vs baseline: 2.0925x; 2.0925x over previous
import jax
import jax.numpy as jnp
from jax import lax
from jax.experimental import pallas as pl

_DIM = 2048
_NE = 64
_TILE = 512


def _gate_body(x_ref, w_ref, b_ref, idx_ref, scr_ref):
    x = x_ref[...]
    w = w_ref[...]
    b = b_ref[...]
    logits = lax.dot_general(w, x, (((1,), (1,)), ((), ())),
                             preferred_element_type=jnp.float32) + b
    eid = lax.broadcasted_iota(jnp.int32, logits.shape, 0)
    m1 = jnp.max(logits, axis=0, keepdims=True)
    i1 = jnp.min(jnp.where(logits == m1, eid, _NE), axis=0, keepdims=True)
    masked = jnp.where(eid == i1, -jnp.inf, logits)
    m2 = jnp.max(masked, axis=0, keepdims=True)
    i2 = jnp.min(jnp.where(masked == m2, eid, _NE), axis=0, keepdims=True)
    s1 = 1.0 / (1.0 + jnp.exp(m2 - m1))
    idx_ref[...] = jnp.concatenate([i1, i2], axis=0)
    scr_ref[...] = jnp.concatenate([s1, 1.0 - s1], axis=0)


def kernel(x, W, b):
    bsz, seq, dim = x.shape
    n_tok = bsz * seq
    x2 = x.reshape(n_tok, dim)
    b2 = b.reshape(_NE, 1)
    grid = (n_tok // _TILE,)
    idx_t, scr_t = pl.pallas_call(
        _gate_body,
        grid=grid,
        in_specs=[
            pl.BlockSpec((_TILE, dim), lambda i: (i, 0)),
            pl.BlockSpec((_NE, dim), lambda i: (0, 0)),
            pl.BlockSpec((_NE, 1), lambda i: (0, 0)),
        ],
        out_specs=[
            pl.BlockSpec((2, _TILE), lambda i: (0, i)),
            pl.BlockSpec((2, _TILE), lambda i: (0, i)),
        ],
        out_shape=[
            jax.ShapeDtypeStruct((2, n_tok), jnp.int32),
            jax.ShapeDtypeStruct((2, n_tok), jnp.float32),
        ],
    )(x2, W, b2)
    idx = idx_t.T.reshape(bsz, seq, 2)
    scr = scr_t.T.reshape(bsz, seq, 2)
    return (idx, scr)


# speedup vs baseline: 2.4851x; 1.1876x over previous
import jax
import jax.numpy as jnp
from jax import lax
from jax.experimental import pallas as pl

_DIM = 2048
_NE = 64
_TILE = 1024


def _gate_body(x_ref, w_ref, b_ref, idx_ref, scr_ref):
    x = x_ref[...]
    w = w_ref[...]
    b = b_ref[...]
    logits = lax.dot_general(w, x, (((1,), (1,)), ((), ())),
                             preferred_element_type=jnp.float32) + b
    eid = lax.broadcasted_iota(jnp.int32, logits.shape, 0)
    m1 = jnp.max(logits, axis=0, keepdims=True)
    i1 = jnp.min(jnp.where(logits == m1, eid, _NE), axis=0, keepdims=True)
    masked = jnp.where(eid == i1, -jnp.inf, logits)
    m2 = jnp.max(masked, axis=0, keepdims=True)
    i2 = jnp.min(jnp.where(masked == m2, eid, _NE), axis=0, keepdims=True)
    s1 = 1.0 / (1.0 + jnp.exp(m2 - m1))
    idx_ref[...] = jnp.concatenate([i1, i2], axis=0)
    scr_ref[...] = jnp.concatenate([s1, 1.0 - s1], axis=0)


def kernel(x, W, b):
    bsz, seq, dim = x.shape
    n_tok = bsz * seq
    x2 = x.reshape(n_tok, dim)
    b2 = b.reshape(_NE, 1)
    grid = (n_tok // _TILE,)
    idx_t, scr_t = pl.pallas_call(
        _gate_body,
        grid=grid,
        in_specs=[
            pl.BlockSpec((_TILE, dim), lambda i: (i, 0)),
            pl.BlockSpec((_NE, dim), lambda i: (0, 0)),
            pl.BlockSpec((_NE, 1), lambda i: (0, 0)),
        ],
        out_specs=[
            pl.BlockSpec((2, _TILE), lambda i: (0, i)),
            pl.BlockSpec((2, _TILE), lambda i: (0, i)),
        ],
        out_shape=[
            jax.ShapeDtypeStruct((2, n_tok), jnp.int32),
            jax.ShapeDtypeStruct((2, n_tok), jnp.float32),
        ],
    )(x2, W, b2)
    idx = idx_t.T.reshape(bsz, seq, 2)
    scr = scr_t.T.reshape(bsz, seq, 2)
    return (idx, scr)
